# Initial kernel scaffold; baseline (speedup 1.0000x reference)
import jax
import jax.numpy as jnp
from jax.experimental import pallas as pl


def kernel(x0, t, noise):
    raise NotImplementedError("write your pallas kernel here")


# speedup vs baseline: 2.1026x; 2.1026x over previous
import functools

import numpy as np
import jax
import jax.numpy as jnp
from jax import lax
from jax.experimental import pallas as pl
from jax.experimental.pallas import tpu as pltpu
from jax.experimental.pallas import tpu_sc as plsc

_N = 16384
_D = 128
_TBL = 1008
_NC, _NS = 2, 16
_NW = _NC * _NS
_BPW = _N // _NW
_LANES = 16


def _make_tables():
    betas = np.concatenate([
        np.zeros((1,), np.float32),
        np.linspace(1e-4, 0.02, 1000, dtype=np.float32),
    ]).astype(np.float32)
    abar = np.cumprod((1.0 - betas).astype(np.float32), dtype=np.float32)
    sqrt_a = np.sqrt(abar).astype(np.float32)
    sqrt_b = np.sqrt((1.0 - abar).astype(np.float32)).astype(np.float32)
    pad = _TBL - abar.shape[0]
    return (np.pad(sqrt_a, (0, pad)), np.pad(sqrt_b, (0, pad)))


_SQRT_A_TBL, _SQRT_B_TBL = _make_tables()


def _sc_gather(t):
    mesh = plsc.VectorSubcoreMesh(core_axis_name="c", subcore_axis_name="s")

    @functools.partial(
        pl.kernel,
        mesh=mesh,
        out_type=[
            jax.ShapeDtypeStruct((_N,), jnp.float32),
            jax.ShapeDtypeStruct((_N,), jnp.float32),
        ],
        scratch_types=[
            pltpu.VMEM((_BPW,), jnp.int32),
            pltpu.VMEM((_TBL,), jnp.float32),
            pltpu.VMEM((_TBL,), jnp.float32),
            pltpu.VMEM((_BPW,), jnp.float32),
            pltpu.VMEM((_BPW,), jnp.float32),
        ],
        compiler_params=pltpu.CompilerParams(needs_layout_passes=False),
    )
    def k(t_hbm, ta_hbm, tb_hbm, oa_hbm, ob_hbm, t_v, ta_v, tb_v, oa_v, ob_v):
        wid = lax.axis_index("s") * _NC + lax.axis_index("c")
        base = wid * _BPW
        pltpu.sync_copy(t_hbm.at[pl.ds(base, _BPW)], t_v)
        pltpu.sync_copy(ta_hbm, ta_v)
        pltpu.sync_copy(tb_hbm, tb_v)

        def body(g, carry):
            off = g * _LANES
            idx = t_v[pl.ds(off, _LANES)]
            oa_v[pl.ds(off, _LANES)] = plsc.load_gather(ta_v, [idx])
            ob_v[pl.ds(off, _LANES)] = plsc.load_gather(tb_v, [idx])
            return carry

        lax.fori_loop(0, _BPW // _LANES, body, 0)
        pltpu.sync_copy(oa_v, oa_hbm.at[pl.ds(base, _BPW)])
        pltpu.sync_copy(ob_v, ob_hbm.at[pl.ds(base, _BPW)])

    return k(t, jnp.asarray(_SQRT_A_TBL), jnp.asarray(_SQRT_B_TBL))


_ROWS_PER_BLOCK = 1024


def _tc_combine(a, b, x0, noise):
    def body(a_ref, b_ref, x_ref, n_ref, o_ref):
        o_ref[...] = b_ref[...] * n_ref[...] + a_ref[...] * x_ref[...]

    grid = _N // _ROWS_PER_BLOCK
    return pl.pallas_call(
        body,
        grid=(grid,),
        in_specs=[
            pl.BlockSpec((_ROWS_PER_BLOCK, 1), lambda i: (i, 0)),
            pl.BlockSpec((_ROWS_PER_BLOCK, 1), lambda i: (i, 0)),
            pl.BlockSpec((_ROWS_PER_BLOCK, _D), lambda i: (i, 0)),
            pl.BlockSpec((_ROWS_PER_BLOCK, _D), lambda i: (i, 0)),
        ],
        out_specs=pl.BlockSpec((_ROWS_PER_BLOCK, _D), lambda i: (i, 0)),
        out_shape=jax.ShapeDtypeStruct((_N, _D), jnp.float32),
    )(a, b, x0, noise)


def kernel(x0, t, noise):
    a, b = _sc_gather(t.astype(jnp.int32))
    return _tc_combine(a.reshape(_N, 1), b.reshape(_N, 1), x0, noise)


# speedup vs baseline: 2.3026x; 1.0951x over previous
import functools

import numpy as np
import jax
import jax.numpy as jnp
from jax import lax
from jax.experimental import pallas as pl
from jax.experimental.pallas import tpu as pltpu
from jax.experimental.pallas import tpu_sc as plsc

_N = 16384
_D = 128
_TBL = 1008
_NC, _NS = 2, 16
_NW = _NC * _NS
_BPW = _N // _NW
_CH = 64
_NCHUNK = _BPW // _CH
_LANES = 16


def _make_tables():
    betas = np.concatenate([
        np.zeros((1,), np.float32),
        np.linspace(1e-4, 0.02, 1000, dtype=np.float32),
    ]).astype(np.float32)
    abar = np.cumprod((1.0 - betas).astype(np.float32), dtype=np.float32)
    sqrt_a = np.sqrt(abar).astype(np.float32)
    sqrt_b = np.sqrt((1.0 - abar).astype(np.float32)).astype(np.float32)
    pad = _TBL - abar.shape[0]
    return (np.pad(sqrt_a, (0, pad)), np.pad(sqrt_b, (0, pad)))


_SQRT_A_TBL, _SQRT_B_TBL = _make_tables()


def _ddpm_sc(x0, t, noise):
    mesh = plsc.VectorSubcoreMesh(core_axis_name="c", subcore_axis_name="s")

    @functools.partial(
        pl.kernel,
        mesh=mesh,
        out_type=jax.ShapeDtypeStruct((_N, _D), jnp.float32),
        scratch_types=[
            pltpu.VMEM((_BPW,), jnp.int32),
            pltpu.VMEM((_TBL,), jnp.float32),
            pltpu.VMEM((_TBL,), jnp.float32),
            pltpu.VMEM((_BPW,), jnp.float32),
            pltpu.VMEM((_BPW,), jnp.float32),
            pltpu.VMEM((2, _CH, _D), jnp.float32),
            pltpu.VMEM((2, _CH, _D), jnp.float32),
            pltpu.VMEM((2, _CH, _D), jnp.float32),
            pltpu.SemaphoreType.DMA,
            pltpu.SemaphoreType.DMA,
            pltpu.SemaphoreType.DMA,
        ],
        compiler_params=pltpu.CompilerParams(needs_layout_passes=False),
    )
    def k(x0_hbm, t_hbm, n_hbm, ta_hbm, tb_hbm, y_hbm,
          t_v, ta_v, tb_v, ca_v, cb_v, x_v, n_v, y_v,
          sem_x, sem_n, sem_y):
        wid = lax.axis_index("s") * _NC + lax.axis_index("c")
        base = wid * _BPW

        pltpu.sync_copy(t_hbm.at[pl.ds(base, _BPW)], t_v)
        pltpu.sync_copy(ta_hbm, ta_v)
        pltpu.sync_copy(tb_hbm, tb_v)

        def gbody(g, carry):
            off = g * _LANES
            idx = t_v[pl.ds(off, _LANES)]
            ca_v[pl.ds(off, _LANES)] = plsc.load_gather(ta_v, [idx])
            cb_v[pl.ds(off, _LANES)] = plsc.load_gather(tb_v, [idx])
            return carry

        lax.fori_loop(0, _BPW // _LANES, gbody, 0)

        def start_in(g):
            b = g % 2
            r0 = base + g * _CH
            hx = pltpu.async_copy(x0_hbm.at[pl.ds(r0, _CH)], x_v.at[b], sem_x)
            hn = pltpu.async_copy(n_hbm.at[pl.ds(r0, _CH)], n_v.at[b], sem_n)
            return hx, hn

        pend = start_in(0)
        out_pend = {}
        for g in range(_NCHUNK):
            b = g % 2
            nxt = start_in(g + 1) if g + 1 < _NCHUNK else None
            for h in pend:
                h.wait()
            if g - 2 in out_pend:
                out_pend.pop(g - 2).wait()

            crow = g * _CH

            def cbody(blk, carry):
                row0 = blk * _LANES
                cav = ca_v[pl.ds(crow + row0, _LANES)]
                cbv = cb_v[pl.ds(crow + row0, _LANES)]
                for k2 in range(_LANES):
                    sav = jnp.full((_LANES,), cav[k2], jnp.float32)
                    sbv = jnp.full((_LANES,), cbv[k2], jnp.float32)
                    r = row0 + k2
                    for j in range(_D // _LANES):
                        sl = pl.ds(j * _LANES, _LANES)
                        y_v[b, r, sl] = (sav * x_v[b, r, sl]
                                         + sbv * n_v[b, r, sl])
                return carry

            lax.fori_loop(0, _CH // _LANES, cbody, 0)

            out_pend[g] = pltpu.async_copy(
                y_v.at[b], y_hbm.at[pl.ds(base + crow, _CH)], sem_y)
            pend = nxt

        for g in sorted(out_pend):
            out_pend.pop(g).wait()

    return k(x0, t.astype(jnp.int32), noise,
             jnp.asarray(_SQRT_A_TBL), jnp.asarray(_SQRT_B_TBL))


def kernel(x0, t, noise):
    return _ddpm_sc(x0, t, noise)


# speedup vs baseline: 2.7735x; 1.2045x over previous
import functools

import numpy as np
import jax
import jax.numpy as jnp
from jax import lax
from jax.experimental import pallas as pl
from jax.experimental.pallas import tpu as pltpu
from jax.experimental.pallas import tpu_sc as plsc

_N = 16384
_D = 128
_TBL = 1008
_NC, _NS = 2, 16
_NW = _NC * _NS
_BPW = _N // _NW
_LANES = 16


def _make_tables():
    betas = np.concatenate([
        np.zeros((1,), np.float32),
        np.linspace(1e-4, 0.02, 1000, dtype=np.float32),
    ]).astype(np.float32)
    abar = np.cumprod((1.0 - betas).astype(np.float32), dtype=np.float32)
    sqrt_a = np.sqrt(abar).astype(np.float32)
    sqrt_b = np.sqrt((1.0 - abar).astype(np.float32)).astype(np.float32)
    pad = _TBL - abar.shape[0]
    return (np.pad(sqrt_a, (0, pad)), np.pad(sqrt_b, (0, pad)))


_SQRT_A_TBL, _SQRT_B_TBL = _make_tables()


def _sc_gather(t):
    mesh = plsc.VectorSubcoreMesh(core_axis_name="c", subcore_axis_name="s")

    @functools.partial(
        pl.kernel,
        mesh=mesh,
        out_type=[
            jax.ShapeDtypeStruct((_N,), jnp.float32),
            jax.ShapeDtypeStruct((_N,), jnp.float32),
        ],
        scratch_types=[
            pltpu.VMEM((_BPW,), jnp.int32),
            pltpu.VMEM((_TBL,), jnp.float32),
            pltpu.VMEM((_TBL,), jnp.float32),
            pltpu.VMEM((_BPW,), jnp.float32),
            pltpu.VMEM((_BPW,), jnp.float32),
        ],
        compiler_params=pltpu.CompilerParams(needs_layout_passes=False),
    )
    def k(t_hbm, ta_hbm, tb_hbm, oa_hbm, ob_hbm, t_v, ta_v, tb_v, oa_v, ob_v):
        wid = lax.axis_index("s") * _NC + lax.axis_index("c")
        base = wid * _BPW
        pltpu.sync_copy(t_hbm.at[pl.ds(base, _BPW)], t_v)
        pltpu.sync_copy(ta_hbm, ta_v)
        pltpu.sync_copy(tb_hbm, tb_v)

        def body(g, carry):
            off = g * _LANES
            idx = t_v[pl.ds(off, _LANES)]
            oa_v[pl.ds(off, _LANES)] = plsc.load_gather(ta_v, [idx])
            ob_v[pl.ds(off, _LANES)] = plsc.load_gather(tb_v, [idx])
            return carry

        lax.fori_loop(0, _BPW // _LANES, body, 0)
        pltpu.sync_copy(oa_v, oa_hbm.at[pl.ds(base, _BPW)])
        pltpu.sync_copy(ob_v, ob_hbm.at[pl.ds(base, _BPW)])

    return k(t, jnp.asarray(_SQRT_A_TBL), jnp.asarray(_SQRT_B_TBL))


_ROWS_PER_BLOCK = 1024


def _tc_combine(a, b, x0, noise):
    def body(a_ref, b_ref, x_ref, n_ref, o_ref):
        av = a_ref[...].reshape(_ROWS_PER_BLOCK, 1)
        bv = b_ref[...].reshape(_ROWS_PER_BLOCK, 1)
        o_ref[...] = bv * n_ref[...] + av * x_ref[...]

    grid = _N // _ROWS_PER_BLOCK
    return pl.pallas_call(
        body,
        grid=(grid,),
        in_specs=[
            pl.BlockSpec((_ROWS_PER_BLOCK,), lambda i: (i,)),
            pl.BlockSpec((_ROWS_PER_BLOCK,), lambda i: (i,)),
            pl.BlockSpec((_ROWS_PER_BLOCK, _D), lambda i: (i, 0)),
            pl.BlockSpec((_ROWS_PER_BLOCK, _D), lambda i: (i, 0)),
        ],
        out_specs=pl.BlockSpec((_ROWS_PER_BLOCK, _D), lambda i: (i, 0)),
        out_shape=jax.ShapeDtypeStruct((_N, _D), jnp.float32),
    )(a, b, x0, noise)


def kernel(x0, t, noise):
    a, b = _sc_gather(t.astype(jnp.int32))
    return _tc_combine(a, b, x0, noise)


# speedup vs baseline: 2.9140x; 1.0506x over previous
import functools

import numpy as np
import jax
import jax.numpy as jnp
from jax import lax
from jax.experimental import pallas as pl
from jax.experimental.pallas import tpu as pltpu
from jax.experimental.pallas import tpu_sc as plsc

_N = 16384
_D = 128
_TBL = 1008
_NC, _NS = 2, 16
_NW = _NC * _NS
_BPW = _N // _NW
_LANES = 16


def _make_tables():
    betas = np.concatenate([
        np.zeros((1,), np.float32),
        np.linspace(1e-4, 0.02, 1000, dtype=np.float32),
    ]).astype(np.float32)
    abar = np.cumprod((1.0 - betas).astype(np.float32), dtype=np.float32)
    sqrt_a = np.sqrt(abar).astype(np.float32)
    sqrt_b = np.sqrt((1.0 - abar).astype(np.float32)).astype(np.float32)
    pad = _TBL - abar.shape[0]
    return (np.pad(sqrt_a, (0, pad)), np.pad(sqrt_b, (0, pad)))


_SQRT_A_TBL, _SQRT_B_TBL = _make_tables()


_GW = _NS
_GBPW = _N // _GW


def _sc_gather(t):
    mesh = plsc.VectorSubcoreMesh(
        core_axis_name="c", subcore_axis_name="s", num_cores=1)

    @functools.partial(
        pl.kernel,
        mesh=mesh,
        out_type=[
            jax.ShapeDtypeStruct((_N,), jnp.float32),
            jax.ShapeDtypeStruct((_N,), jnp.float32),
        ],
        scratch_types=[
            pltpu.VMEM((_GBPW,), jnp.int32),
            pltpu.VMEM((_TBL,), jnp.float32),
            pltpu.VMEM((_TBL,), jnp.float32),
            pltpu.VMEM((_GBPW,), jnp.float32),
            pltpu.VMEM((_GBPW,), jnp.float32),
        ],
        compiler_params=pltpu.CompilerParams(needs_layout_passes=False),
    )
    def k(t_hbm, ta_hbm, tb_hbm, oa_hbm, ob_hbm, t_v, ta_v, tb_v, oa_v, ob_v):
        wid = lax.axis_index("s")
        base = wid * _GBPW
        pltpu.sync_copy(t_hbm.at[pl.ds(base, _GBPW)], t_v)
        pltpu.sync_copy(ta_hbm, ta_v)
        pltpu.sync_copy(tb_hbm, tb_v)

        def body(g, carry):
            off = g * _LANES
            idx = t_v[pl.ds(off, _LANES)]
            oa_v[pl.ds(off, _LANES)] = plsc.load_gather(ta_v, [idx])
            ob_v[pl.ds(off, _LANES)] = plsc.load_gather(tb_v, [idx])
            return carry

        lax.fori_loop(0, _GBPW // _LANES, body, 0)
        pltpu.sync_copy(oa_v, oa_hbm.at[pl.ds(base, _GBPW)])
        pltpu.sync_copy(ob_v, ob_hbm.at[pl.ds(base, _GBPW)])

    return k(t, jnp.asarray(_SQRT_A_TBL), jnp.asarray(_SQRT_B_TBL))


_ROWS_PER_BLOCK = 1024


def _tc_combine(a, b, x0, noise):
    def body(a_ref, b_ref, x_ref, n_ref, o_ref):
        av = a_ref[...].reshape(_ROWS_PER_BLOCK, 1)
        bv = b_ref[...].reshape(_ROWS_PER_BLOCK, 1)
        o_ref[...] = bv * n_ref[...] + av * x_ref[...]

    grid = _N // _ROWS_PER_BLOCK
    return pl.pallas_call(
        body,
        grid=(grid,),
        in_specs=[
            pl.BlockSpec((_ROWS_PER_BLOCK,), lambda i: (i,)),
            pl.BlockSpec((_ROWS_PER_BLOCK,), lambda i: (i,)),
            pl.BlockSpec((_ROWS_PER_BLOCK, _D), lambda i: (i, 0)),
            pl.BlockSpec((_ROWS_PER_BLOCK, _D), lambda i: (i, 0)),
        ],
        out_specs=pl.BlockSpec((_ROWS_PER_BLOCK, _D), lambda i: (i, 0)),
        out_shape=jax.ShapeDtypeStruct((_N, _D), jnp.float32),
    )(a, b, x0, noise)


def kernel(x0, t, noise):
    a, b = _sc_gather(t.astype(jnp.int32))
    return _tc_combine(a, b, x0, noise)
